# Initial kernel scaffold; baseline (speedup 1.0000x reference)
import jax
import jax.numpy as jnp
from jax.experimental import pallas as pl


def kernel(x_user, x_item, edge_index_rates, edge_index_rated_by, edge_index_follows, edge_index_similar, lin_user_W, lin_user_b, lin_item_W, lin_item_b, basis_0, comp_0, root_0, bias_0, basis_1, comp_1, root_1, bias_1):
    raise NotImplementedError("write your pallas kernel here")


# speedup vs baseline: 3.6143x; 3.6143x over previous
import functools

import jax
import jax.numpy as jnp
from jax import lax
from jax.experimental import pallas as pl
from jax.experimental.pallas import tpu as pltpu
from jax.experimental.pallas import tpu_sc as plsc

N_U = 25000
N_ALL = 2 * N_U
HF = 128
NREL = 4
NE = 150000
CHUNK = 128
NCHUNK = 1184
E_PAD = NCHUNK * CHUNK
NTILE = 16
NCORE = 2
CPT = NCHUNK // NTILE
HALF = 12544
ACC_ROWS = 12800
OUT_ROWS = 2 * HALF
RPT = HALF // NTILE
WB = 112
ZCH = 32
CNT_ROWS = 25600
ROWB = 1000
NBLK_U = N_U // ROWB


@functools.cache
def _sc_mesh():
  return plsc.VectorSubcoreMesh(core_axis_name="c", subcore_axis_name="s",
                                num_cores=NCORE, num_subcores=NTILE)


def _zero_fill(ref, rows, cols):
  zv = jnp.zeros((16,), jnp.float32)
  per_row = cols // 16

  def body(k, _):
    i = k // per_row
    j = k % per_row
    ref[i, pl.ds(j * 16, 16)] = zv
    return 0

  lax.fori_loop(0, rows * per_row, body, 0)


def _zero_fill_1d(ref, n, value=0.0):
  v = jnp.full((16,), value, jnp.float32)

  def body(k, _):
    ref[pl.ds(k * 16, 16)] = v
    return 0

  lax.fori_loop(0, n // 16, body, 0)


def _count_body(dst_hbm, cnt_hbm, idx_v, ones_v, zb_v, acc):
  c = lax.axis_index("c")
  t = lax.axis_index("s")

  _zero_fill_1d(zb_v, 3200, 0.0)
  _zero_fill_1d(ones_v, CHUNK, 1.0)

  pltpu.sync_copy(zb_v, acc.at[pl.ds(pl.multiple_of(t * 3200, 8), 3200)])
  plsc.subcore_barrier()

  for r_local in range(2):
    rel_off = r_local * CNT_ROWS

    def chunk_body(j, _):
      tile_off = pl.multiple_of((t * CPT + j) * CHUNK, CHUNK)
      for r_glob in range(NREL):
        @pl.when(c * 2 + r_local == r_glob)
        def _():
          pltpu.sync_copy(
              dst_hbm.at[pl.ds(r_glob * E_PAD + tile_off, CHUNK)], idx_v)

      def shift(k, _):
        idx_v[pl.ds(k * 16, 16)] = idx_v[pl.ds(k * 16, 16)] + rel_off
        return 0

      lax.fori_loop(0, CHUNK // 16, shift, 0)
      pltpu.sync_copy(ones_v, acc.at[idx_v], add=True)
      return 0

    lax.fori_loop(0, CPT, chunk_body, 0)

  plsc.subcore_barrier()
  for r_local in range(2):
    toff = pl.multiple_of(t * 1600, 8)
    pltpu.sync_copy(
        acc.at[pl.ds(r_local * CNT_ROWS + toff, 1600)],
        zb_v.at[pl.ds(0, 1600)],
    )
    for r_glob in range(NREL):
      @pl.when(c * 2 + r_local == r_glob)
      def _():
        pltpu.sync_copy(
            zb_v.at[pl.ds(0, 1600)],
            cnt_hbm.at[pl.ds(r_glob * CNT_ROWS + toff, 1600)],
        )


@functools.cache
def _count_kernel_built():
  return pl.kernel(
      _count_body,
      out_type=jax.ShapeDtypeStruct((NREL * CNT_ROWS,), jnp.float32),
      mesh=_sc_mesh(),
      scratch_types=[
          pltpu.VMEM((CHUNK,), jnp.int32),
          pltpu.VMEM((CHUNK,), jnp.float32),
          pltpu.VMEM((3200,), jnp.float32),
          pltpu.VMEM_SHARED((2 * CNT_ROWS,), jnp.float32),
      ],
  )


def _count_kernel(dst_g):
  return _count_kernel_built()(dst_g).reshape(NREL, CNT_ROWS)


def _segsum_body(h_hbm, src_hbm, dst_hbm, a_hbm,
                 src_v, dst_v, rows_v, zb_v, sem, acc):
  c = lax.axis_index("c")
  t = lax.axis_index("s")
  base = c * HALF

  _zero_fill(zb_v, ZCH, HF)

  for r in range(NREL):
    for z in range(800 // ZCH):
      pltpu.sync_copy(
          zb_v, acc.at[pl.ds(pl.multiple_of(t * 800 + z * ZCH, 8), ZCH)])
    plsc.subcore_barrier()

    def chunk_body(j, _):
      off = pl.multiple_of((t * CPT + j) * CHUNK, CHUNK)
      pltpu.sync_copy(src_hbm.at[pl.ds(r * E_PAD + off, CHUNK)], src_v)
      pltpu.sync_copy(dst_hbm.at[pl.ds(r * E_PAD + off, CHUNK)], dst_v)

      def remap(k, _):
        d = dst_v[pl.ds(k * 16, 16)]
        local = d - base
        ok = (local >= 0) & (local < HALF)
        trash = HALF + (d & 255)
        dst_v[pl.ds(k * 16, 16)] = jnp.where(ok, local, trash)
        return 0

      lax.fori_loop(0, CHUNK // 16, remap, 0)

      pltpu.async_copy(h_hbm.at[src_v], rows_v, sem).wait()
      pltpu.sync_copy(rows_v, acc.at[dst_v], add=True)
      return 0

    lax.fori_loop(0, CPT, chunk_body, 0)
    plsc.subcore_barrier()

    for w in range(RPT // WB):
      row0 = pl.multiple_of(t * RPT + w * WB, 8)
      pltpu.sync_copy(acc.at[pl.ds(row0, WB)], rows_v.at[pl.ds(0, WB)])
      pltpu.sync_copy(rows_v.at[pl.ds(0, WB)],
                      a_hbm.at[r, pl.ds(pl.multiple_of(base + row0, 8), WB)])

    plsc.subcore_barrier()


@functools.cache
def _segsum_kernel_built():
  return pl.kernel(
      _segsum_body,
      out_type=jax.ShapeDtypeStruct((NREL, OUT_ROWS, HF), jnp.float32),
      mesh=_sc_mesh(),
      scratch_types=[
          pltpu.VMEM((CHUNK,), jnp.int32),
          pltpu.VMEM((CHUNK,), jnp.int32),
          pltpu.VMEM((CHUNK, HF), jnp.float32),
          pltpu.VMEM((ZCH, HF), jnp.float32),
          pltpu.SemaphoreType.DMA,
          pltpu.VMEM_SHARED((ACC_ROWS, HF), jnp.float32),
      ],
  )


def _segsum_kernel(h, src_g, dst_g):
  return _segsum_kernel_built()(h, src_g, dst_g)


def _proj_body(x_ref, w_ref, b_ref, o_ref):
  h = jnp.dot(x_ref[...], w_ref[...], preferred_element_type=jnp.float32)
  o_ref[...] = h + b_ref[...]


def _project(x, w, b):
  n, k = x.shape
  grid = n // ROWB
  return pl.pallas_call(
      _proj_body,
      grid=(grid,),
      in_specs=[
          pl.BlockSpec((ROWB, k), lambda i: (i, 0)),
          pl.BlockSpec((k, HF), lambda i: (0, 0)),
          pl.BlockSpec((1, HF), lambda i: (0, 0)),
      ],
      out_specs=pl.BlockSpec((ROWB, HF), lambda i: (i, 0)),
      out_shape=jax.ShapeDtypeStruct((n, HF), jnp.float32),
  )(x, w, b.reshape(1, HF))


def _layer_body(h_ref, a1_ref, a2_ref, c1_ref, c2_ref,
                root_ref, b1_ref, b2_ref, bias_ref, o_ref):
  acc = jnp.dot(h_ref[...], root_ref[...], preferred_element_type=jnp.float32)

  r1 = 1.0 / jnp.maximum(c1_ref[0, 0], 1.0)
  r2 = 1.0 / jnp.maximum(c2_ref[0, 0], 1.0)

  acc += jnp.dot(a1_ref[0] * r1[:, None], b1_ref[0],
                 preferred_element_type=jnp.float32)
  acc += jnp.dot(a2_ref[0] * r2[:, None], b2_ref[0],
                 preferred_element_type=jnp.float32)

  acc += bias_ref[...]
  o_ref[...] = jnp.maximum(acc, 0.0)


def _layer(h, a, cnt, root, b1, b2, bias):
  grid = N_ALL // ROWB
  cnt_r = cnt[:, :N_U].reshape(NREL * NBLK_U, 1, ROWB)

  def iu(i):
    return jnp.where(i < NBLK_U, i, i - NBLK_U)

  def rel1(i):
    return jnp.where(i < NBLK_U, 1, 0)

  def rel2(i):
    return jnp.where(i < NBLK_U, 2, 3)

  def dom(i):
    return jnp.where(i < NBLK_U, 0, 1)

  return pl.pallas_call(
      _layer_body,
      grid=(grid,),
      in_specs=[
          pl.BlockSpec((ROWB, HF), lambda i: (i, 0)),
          pl.BlockSpec((1, ROWB, HF), lambda i: (rel1(i), iu(i), 0)),
          pl.BlockSpec((1, ROWB, HF), lambda i: (rel2(i), iu(i), 0)),
          pl.BlockSpec((1, 1, ROWB), lambda i: (rel1(i) * NBLK_U + iu(i), 0, 0)),
          pl.BlockSpec((1, 1, ROWB), lambda i: (rel2(i) * NBLK_U + iu(i), 0, 0)),
          pl.BlockSpec((HF, HF), lambda i: (0, 0)),
          pl.BlockSpec((1, HF, HF), lambda i: (dom(i), 0, 0)),
          pl.BlockSpec((1, HF, HF), lambda i: (dom(i), 0, 0)),
          pl.BlockSpec((1, HF), lambda i: (0, 0)),
      ],
      out_specs=pl.BlockSpec((ROWB, HF), lambda i: (i, 0)),
      out_shape=jax.ShapeDtypeStruct((N_ALL, HF), jnp.float32),
  )(h, a, a, cnt_r, cnt_r, root, b1, b2, bias.reshape(1, HF))


def kernel(x_user, x_item, edge_index_rates, edge_index_rated_by,
           edge_index_follows, edge_index_similar, lin_user_W, lin_user_b,
           lin_item_W, lin_item_b, basis_0, comp_0, root_0, bias_0,
           basis_1, comp_1, root_1, bias_1):
  pad = E_PAD - NE
  pad_src = (jnp.arange(pad, dtype=jnp.int32) % 64)
  pad_dst = N_U + (jnp.arange(pad, dtype=jnp.int32) % (OUT_ROWS - N_U))

  def prep(ei, src_off):
    s = jnp.concatenate([ei[0] + src_off, pad_src])
    d = jnp.concatenate([ei[1], pad_dst])
    return s, d

  s0, d0 = prep(edge_index_rates, 0)
  s1, d1 = prep(edge_index_rated_by, N_U)
  s2, d2 = prep(edge_index_follows, 0)
  s3, d3 = prep(edge_index_similar, N_U)
  src_g = jnp.concatenate([s0, s1, s2, s3]).astype(jnp.int32)
  dst_g = jnp.concatenate([d0, d1, d2, d3]).astype(jnp.int32)

  hu = _project(x_user, lin_user_W, lin_user_b)
  hi = _project(x_item, lin_item_W, lin_item_b)
  h = jnp.concatenate([hu, hi], axis=0)

  cnt = _count_kernel(dst_g)

  def weights(basis, comp):
    b = basis[0]
    b1 = jnp.stack([comp[1, 0] * b, comp[0, 0] * b])
    b2 = jnp.stack([comp[2, 0] * b, comp[3, 0] * b])
    return b1, b2

  b1_0, b2_0 = weights(basis_0, comp_0)
  b1_1, b2_1 = weights(basis_1, comp_1)

  for root, b1, b2, bias in ((root_0, b1_0, b2_0, bias_0),
                             (root_1, b1_1, b2_1, bias_1)):
    a = _segsum_kernel(h, src_g, dst_g)
    h = _layer(h, a, cnt, root, b1, b2, bias)

  return h[:N_U], h[N_U:]


# speedup vs baseline: 4.8830x; 1.3510x over previous
import functools

import jax
import jax.numpy as jnp
from jax import lax
from jax.experimental import pallas as pl
from jax.experimental.pallas import tpu as pltpu
from jax.experimental.pallas import tpu_sc as plsc

N_U = 25000
N_ALL = 2 * N_U
HF = 128
NREL = 4
NE = 150000
CHUNK = 96
NCHUNK = 1568
E_PAD = NCHUNK * CHUNK
NTILE = 16
NCORE = 2
CPT = NCHUNK // NTILE
HALF = 12544
ACC_ROWS = 12608
OUT_ROWS = 2 * HALF
RPT = HALF // NTILE
WB = 56
ZCH = 16
CNT_ROWS = 25600
ROWB = 1000
NBLK_U = N_U // ROWB


@functools.cache
def _sc_mesh():
  return plsc.VectorSubcoreMesh(core_axis_name="c", subcore_axis_name="s",
                                num_cores=NCORE, num_subcores=NTILE)


def _zero_fill(ref, rows, cols):
  zv = jnp.zeros((16,), jnp.float32)
  per_row = cols // 16

  def body(k, _):
    i = k // per_row
    j = k % per_row
    ref[i, pl.ds(j * 16, 16)] = zv
    return 0

  lax.fori_loop(0, rows * per_row, body, 0)


def _zero_fill_1d(ref, n, value=0.0):
  v = jnp.full((16,), value, jnp.float32)

  def body(k, _):
    ref[pl.ds(k * 16, 16)] = v
    return 0

  lax.fori_loop(0, n // 16, body, 0)


def _count_body(dst_hbm, cnt_hbm, idx_v, ones_v, zb_v, acc):
  c = lax.axis_index("c")
  t = lax.axis_index("s")

  _zero_fill_1d(zb_v, 3200, 0.0)
  _zero_fill_1d(ones_v, CHUNK, 1.0)

  pltpu.sync_copy(zb_v, acc.at[pl.ds(pl.multiple_of(t * 3200, 8), 3200)])
  plsc.subcore_barrier()

  for r_local in range(2):
    rel_off = r_local * CNT_ROWS

    def chunk_body(j, _):
      tile_off = pl.multiple_of((t * CPT + j) * CHUNK, CHUNK)
      for r_glob in range(NREL):
        @pl.when(c * 2 + r_local == r_glob)
        def _():
          pltpu.sync_copy(
              dst_hbm.at[pl.ds(r_glob * E_PAD + tile_off, CHUNK)], idx_v)

      def shift(k, _):
        idx_v[pl.ds(k * 16, 16)] = idx_v[pl.ds(k * 16, 16)] + rel_off
        return 0

      lax.fori_loop(0, CHUNK // 16, shift, 0)
      pltpu.sync_copy(ones_v, acc.at[idx_v], add=True)
      return 0

    lax.fori_loop(0, CPT, chunk_body, 0)

  plsc.subcore_barrier()
  for r_local in range(2):
    toff = pl.multiple_of(t * 1600, 8)
    pltpu.sync_copy(
        acc.at[pl.ds(r_local * CNT_ROWS + toff, 1600)],
        zb_v.at[pl.ds(0, 1600)],
    )
    for r_glob in range(NREL):
      @pl.when(c * 2 + r_local == r_glob)
      def _():
        pltpu.sync_copy(
            zb_v.at[pl.ds(0, 1600)],
            cnt_hbm.at[pl.ds(r_glob * CNT_ROWS + toff, 1600)],
        )


@functools.cache
def _count_kernel_built():
  return pl.kernel(
      _count_body,
      out_type=jax.ShapeDtypeStruct((NREL * CNT_ROWS,), jnp.float32),
      mesh=_sc_mesh(),
      scratch_types=[
          pltpu.VMEM((CHUNK,), jnp.int32),
          pltpu.VMEM((CHUNK,), jnp.float32),
          pltpu.VMEM((3200,), jnp.float32),
          pltpu.VMEM_SHARED((2 * CNT_ROWS,), jnp.float32),
      ],
  )


def _count_kernel(dst_g):
  return _count_kernel_built()(dst_g).reshape(NREL, CNT_ROWS)


def _segsum_body(h_hbm, src_hbm, dst_hbm, a_hbm,
                 src_v, dst_v, rows_v, zb_v, sem0, sem1, acc):
  c = lax.axis_index("c")
  t = lax.axis_index("s")
  base = c * HALF
  sems = (sem0, sem1)

  _zero_fill(zb_v, ZCH, HF)

  for r in range(NREL):
    for z in range(800 // ZCH):
      pltpu.sync_copy(
          zb_v, acc.at[pl.ds(pl.multiple_of(t * 800 + z * ZCH, 8), ZCH)])
    plsc.subcore_barrier()

    def issue(j, b):
      off = pl.multiple_of((t * CPT + j) * CHUNK, CHUNK)
      pltpu.sync_copy(src_hbm.at[pl.ds(r * E_PAD + off, CHUNK)], src_v.at[b])
      pltpu.sync_copy(dst_hbm.at[pl.ds(r * E_PAD + off, CHUNK)], dst_v.at[b])

      def remap(k, _):
        d = dst_v[b, pl.ds(k * 16, 16)]
        local = d - base
        ok = (local >= 0) & (local < HALF)
        trash = HALF + (d & 63)
        dst_v[b, pl.ds(k * 16, 16)] = jnp.where(ok, local, trash)
        return 0

      lax.fori_loop(0, CHUNK // 16, remap, 0)
      pltpu.async_copy(h_hbm.at[src_v.at[b]], rows_v.at[b], sems[b])

    def finish(b):
      pltpu.make_async_copy(h_hbm.at[src_v.at[b]], rows_v.at[b],
                            sems[b]).wait()
      pltpu.sync_copy(rows_v.at[b], acc.at[dst_v.at[b]], add=True)

    issue(0, 0)

    def chunk_body(i2, _):
      j = i2 * 2
      issue(j + 1, 1)
      finish(0)

      @pl.when(j + 2 < CPT)
      def _():
        issue(j + 2, 0)

      finish(1)
      return 0

    lax.fori_loop(0, CPT // 2, chunk_body, 0)
    plsc.subcore_barrier()

    for w in range(RPT // WB):
      row0 = pl.multiple_of(t * RPT + w * WB, 8)
      pltpu.sync_copy(acc.at[pl.ds(row0, WB)], rows_v.at[0, pl.ds(0, WB)])
      pltpu.sync_copy(rows_v.at[0, pl.ds(0, WB)],
                      a_hbm.at[r, pl.ds(pl.multiple_of(base + row0, 8), WB)])

    plsc.subcore_barrier()


@functools.cache
def _segsum_kernel_built():
  return pl.kernel(
      _segsum_body,
      out_type=jax.ShapeDtypeStruct((NREL, OUT_ROWS, HF), jnp.float32),
      mesh=_sc_mesh(),
      scratch_types=[
          pltpu.VMEM((2, CHUNK), jnp.int32),
          pltpu.VMEM((2, CHUNK), jnp.int32),
          pltpu.VMEM((2, CHUNK, HF), jnp.float32),
          pltpu.VMEM((ZCH, HF), jnp.float32),
          pltpu.SemaphoreType.DMA,
          pltpu.SemaphoreType.DMA,
          pltpu.VMEM_SHARED((ACC_ROWS, HF), jnp.float32),
      ],
  )


def _segsum_kernel(h, src_g, dst_g):
  return _segsum_kernel_built()(h, src_g, dst_g)


def _proj_body(x_ref, w_ref, b_ref, o_ref):
  h = jnp.dot(x_ref[...], w_ref[...], preferred_element_type=jnp.float32)
  o_ref[...] = h + b_ref[...]


def _project(x, w, b):
  n, k = x.shape
  grid = n // ROWB
  return pl.pallas_call(
      _proj_body,
      grid=(grid,),
      in_specs=[
          pl.BlockSpec((ROWB, k), lambda i: (i, 0)),
          pl.BlockSpec((k, HF), lambda i: (0, 0)),
          pl.BlockSpec((1, HF), lambda i: (0, 0)),
      ],
      out_specs=pl.BlockSpec((ROWB, HF), lambda i: (i, 0)),
      out_shape=jax.ShapeDtypeStruct((n, HF), jnp.float32),
  )(x, w, b.reshape(1, HF))


def _layer_body(h_ref, a1_ref, a2_ref, c1_ref, c2_ref,
                root_ref, b1_ref, b2_ref, bias_ref, o_ref):
  acc = jnp.dot(h_ref[...], root_ref[...], preferred_element_type=jnp.float32)

  r1 = 1.0 / jnp.maximum(c1_ref[0, 0], 1.0)
  r2 = 1.0 / jnp.maximum(c2_ref[0, 0], 1.0)

  acc += jnp.dot(a1_ref[0] * r1[:, None], b1_ref[0],
                 preferred_element_type=jnp.float32)
  acc += jnp.dot(a2_ref[0] * r2[:, None], b2_ref[0],
                 preferred_element_type=jnp.float32)

  acc += bias_ref[...]
  o_ref[...] = jnp.maximum(acc, 0.0)


def _layer(h, a, cnt, root, b1, b2, bias):
  grid = N_ALL // ROWB
  cnt_r = cnt[:, :N_U].reshape(NREL * NBLK_U, 1, ROWB)

  def iu(i):
    return jnp.where(i < NBLK_U, i, i - NBLK_U)

  def rel1(i):
    return jnp.where(i < NBLK_U, 1, 0)

  def rel2(i):
    return jnp.where(i < NBLK_U, 2, 3)

  def dom(i):
    return jnp.where(i < NBLK_U, 0, 1)

  return pl.pallas_call(
      _layer_body,
      grid=(grid,),
      in_specs=[
          pl.BlockSpec((ROWB, HF), lambda i: (i, 0)),
          pl.BlockSpec((1, ROWB, HF), lambda i: (rel1(i), iu(i), 0)),
          pl.BlockSpec((1, ROWB, HF), lambda i: (rel2(i), iu(i), 0)),
          pl.BlockSpec((1, 1, ROWB), lambda i: (rel1(i) * NBLK_U + iu(i), 0, 0)),
          pl.BlockSpec((1, 1, ROWB), lambda i: (rel2(i) * NBLK_U + iu(i), 0, 0)),
          pl.BlockSpec((HF, HF), lambda i: (0, 0)),
          pl.BlockSpec((1, HF, HF), lambda i: (dom(i), 0, 0)),
          pl.BlockSpec((1, HF, HF), lambda i: (dom(i), 0, 0)),
          pl.BlockSpec((1, HF), lambda i: (0, 0)),
      ],
      out_specs=pl.BlockSpec((ROWB, HF), lambda i: (i, 0)),
      out_shape=jax.ShapeDtypeStruct((N_ALL, HF), jnp.float32),
  )(h, a, a, cnt_r, cnt_r, root, b1, b2, bias.reshape(1, HF))


def kernel(x_user, x_item, edge_index_rates, edge_index_rated_by,
           edge_index_follows, edge_index_similar, lin_user_W, lin_user_b,
           lin_item_W, lin_item_b, basis_0, comp_0, root_0, bias_0,
           basis_1, comp_1, root_1, bias_1):
  pad = E_PAD - NE
  pad_src = (jnp.arange(pad, dtype=jnp.int32) % 64)
  pad_dst = N_U + (jnp.arange(pad, dtype=jnp.int32) % (OUT_ROWS - N_U))

  def prep(ei, src_off):
    s = jnp.concatenate([ei[0] + src_off, pad_src])
    d = jnp.concatenate([ei[1], pad_dst])
    return s, d

  s0, d0 = prep(edge_index_rates, 0)
  s1, d1 = prep(edge_index_rated_by, N_U)
  s2, d2 = prep(edge_index_follows, 0)
  s3, d3 = prep(edge_index_similar, N_U)
  src_g = jnp.concatenate([s0, s1, s2, s3]).astype(jnp.int32)
  dst_g = jnp.concatenate([d0, d1, d2, d3]).astype(jnp.int32)

  hu = _project(x_user, lin_user_W, lin_user_b)
  hi = _project(x_item, lin_item_W, lin_item_b)
  h = jnp.concatenate([hu, hi], axis=0)

  cnt = _count_kernel(dst_g)

  def weights(basis, comp):
    b = basis[0]
    b1 = jnp.stack([comp[1, 0] * b, comp[0, 0] * b])
    b2 = jnp.stack([comp[2, 0] * b, comp[3, 0] * b])
    return b1, b2

  b1_0, b2_0 = weights(basis_0, comp_0)
  b1_1, b2_1 = weights(basis_1, comp_1)

  for root, b1, b2, bias in ((root_0, b1_0, b2_0, bias_0),
                             (root_1, b1_1, b2_1, bias_1)):
    a = _segsum_kernel(h, src_g, dst_g)
    h = _layer(h, a, cnt, root, b1, b2, bias)

  return h[:N_U], h[N_U:]


# speedup vs baseline: 6.5663x; 1.3447x over previous
import functools

import jax
import jax.numpy as jnp
from jax import lax
from jax.experimental import pallas as pl
from jax.experimental.pallas import tpu as pltpu
from jax.experimental.pallas import tpu_sc as plsc

N_U = 25000
N_ALL = 2 * N_U
HF = 128
NREL = 4
NE = 150000
CHUNK = 96
NCHUNK = 1568
E_PAD = NCHUNK * CHUNK
NTILE = 16
NCORE = 2
CPT = NCHUNK // NTILE
HALF = 12544
ACC_ROWS = 12608
OUT_ROWS = 2 * HALF
RPT = HALF // NTILE
WB = 56
ZCH = 16
CNT_ROWS = 25600
ROWB = 1000
NBLK_U = N_U // ROWB


@functools.cache
def _sc_mesh():
  return plsc.VectorSubcoreMesh(core_axis_name="c", subcore_axis_name="s",
                                num_cores=NCORE, num_subcores=NTILE)


def _zero_fill(ref, rows, cols):
  zv = jnp.zeros((16,), jnp.float32)
  per_row = cols // 16

  def body(k, _):
    i = k // per_row
    j = k % per_row
    ref[i, pl.ds(j * 16, 16)] = zv
    return 0

  lax.fori_loop(0, rows * per_row, body, 0)


def _zero_fill_1d(ref, n, value=0.0):
  v = jnp.full((16,), value, jnp.float32)

  def body(k, _):
    ref[pl.ds(k * 16, 16)] = v
    return 0

  lax.fori_loop(0, n // 16, body, 0)


def _count_body(dst_hbm, cnt_hbm, idx_v, ones_v, zb_v, acc):
  c = lax.axis_index("c")
  t = lax.axis_index("s")

  _zero_fill_1d(zb_v, 3200, 0.0)
  _zero_fill_1d(ones_v, CHUNK, 1.0)

  pltpu.sync_copy(zb_v, acc.at[pl.ds(pl.multiple_of(t * 3200, 8), 3200)])
  plsc.subcore_barrier()

  for r_local in range(2):
    rel_off = r_local * CNT_ROWS

    def chunk_body(j, _):
      tile_off = pl.multiple_of((t * CPT + j) * CHUNK, CHUNK)
      for r_glob in range(NREL):
        @pl.when(c * 2 + r_local == r_glob)
        def _():
          pltpu.sync_copy(
              dst_hbm.at[pl.ds(r_glob * E_PAD + tile_off, CHUNK)], idx_v)

      def shift(k, _):
        idx_v[pl.ds(k * 16, 16)] = idx_v[pl.ds(k * 16, 16)] + rel_off
        return 0

      lax.fori_loop(0, CHUNK // 16, shift, 0)
      pltpu.sync_copy(ones_v, acc.at[idx_v], add=True)
      return 0

    lax.fori_loop(0, CPT, chunk_body, 0)

  plsc.subcore_barrier()
  for r_local in range(2):
    toff = pl.multiple_of(t * 1600, 8)
    pltpu.sync_copy(
        acc.at[pl.ds(r_local * CNT_ROWS + toff, 1600)],
        zb_v.at[pl.ds(0, 1600)],
    )
    for r_glob in range(NREL):
      @pl.when(c * 2 + r_local == r_glob)
      def _():
        pltpu.sync_copy(
            zb_v.at[pl.ds(0, 1600)],
            cnt_hbm.at[pl.ds(r_glob * CNT_ROWS + toff, 1600)],
        )


@functools.cache
def _count_kernel_built():
  return pl.kernel(
      _count_body,
      out_type=jax.ShapeDtypeStruct((NREL * CNT_ROWS,), jnp.float32),
      mesh=_sc_mesh(),
      scratch_types=[
          pltpu.VMEM((CHUNK,), jnp.int32),
          pltpu.VMEM((CHUNK,), jnp.float32),
          pltpu.VMEM((3200,), jnp.float32),
          pltpu.VMEM_SHARED((2 * CNT_ROWS,), jnp.float32),
      ],
  )


def _count_kernel(dst_g):
  return _count_kernel_built()(dst_g).reshape(NREL, CNT_ROWS)


def _segsum_body(h_hbm, sd_hbm, a_hbm,
                 idx_v, rows_v, zb_v, si0, si1, si2, si3, sg0, sg1, acc):
  c = lax.axis_index("c")
  t = lax.axis_index("s")
  base = c * HALF
  isems = (si0, si1, si2, si3)
  gsems = (sg0, sg1)

  _zero_fill(zb_v, ZCH, HF)

  for r in range(NREL):
    for z in range(800 // ZCH):
      pltpu.sync_copy(
          zb_v, acc.at[pl.ds(pl.multiple_of(t * 800 + z * ZCH, 8), ZCH)])
    plsc.subcore_barrier()

    def idx_start(j):
      chunk_id = r * NCHUNK + t * CPT + j
      for sl in range(4):
        @pl.when((j & 3) == sl)
        def _(sl=sl):
          pltpu.async_copy(sd_hbm.at[chunk_id], idx_v.at[sl], isems[sl])

    def gather_start(j):
      for sl in range(4):
        @pl.when((j & 3) == sl)
        def _(sl=sl):
          chunk_id = r * NCHUNK + t * CPT + j
          pltpu.make_async_copy(sd_hbm.at[chunk_id], idx_v.at[sl],
                                isems[sl]).wait()

          def remap(k, _):
            d = idx_v[sl, 1, pl.ds(k * 16, 16)]
            local = d - base
            ok = (local >= 0) & (local < HALF)
            trash = HALF + (d & 63)
            idx_v[sl, 1, pl.ds(k * 16, 16)] = jnp.where(ok, local, trash)
            return 0

          lax.fori_loop(0, CHUNK // 16, remap, 0)
          for b in range(2):
            @pl.when((j & 1) == b)
            def _(sl=sl, b=b):
              pltpu.async_copy(h_hbm.at[idx_v.at[sl, 0]], rows_v.at[b],
                               gsems[b])

    def finish(j):
      for sl in range(4):
        for b in range(2):
          @pl.when(((j & 3) == sl) & ((j & 1) == b))
          def _(sl=sl, b=b):
            pltpu.make_async_copy(h_hbm.at[idx_v.at[sl, 0]], rows_v.at[b],
                                  gsems[b]).wait()
            pltpu.sync_copy(rows_v.at[b], acc.at[idx_v.at[sl, 1]], add=True)

    idx_start(0)
    idx_start(1)
    idx_start(2)

    def chunk_body(j, _):
      gather_start(j)

      @pl.when(j > 0)
      def _():
        finish(j - 1)

      @pl.when(j + 3 < CPT)
      def _():
        idx_start(j + 3)

      return 0

    lax.fori_loop(0, CPT, chunk_body, 0)
    finish(CPT - 1)
    plsc.subcore_barrier()

    for w in range(RPT // WB):
      row0 = pl.multiple_of(t * RPT + w * WB, 8)
      pltpu.sync_copy(acc.at[pl.ds(row0, WB)], rows_v.at[0, pl.ds(0, WB)])
      pltpu.sync_copy(rows_v.at[0, pl.ds(0, WB)],
                      a_hbm.at[r, pl.ds(pl.multiple_of(base + row0, 8), WB)])

    plsc.subcore_barrier()


@functools.cache
def _segsum_kernel_built():
  return pl.kernel(
      _segsum_body,
      out_type=jax.ShapeDtypeStruct((NREL, OUT_ROWS, HF), jnp.float32),
      mesh=_sc_mesh(),
      scratch_types=[
          pltpu.VMEM((4, 2, CHUNK), jnp.int32),
          pltpu.VMEM((2, CHUNK, HF), jnp.float32),
          pltpu.VMEM((ZCH, HF), jnp.float32),
          pltpu.SemaphoreType.DMA,
          pltpu.SemaphoreType.DMA,
          pltpu.SemaphoreType.DMA,
          pltpu.SemaphoreType.DMA,
          pltpu.SemaphoreType.DMA,
          pltpu.SemaphoreType.DMA,
          pltpu.VMEM_SHARED((ACC_ROWS, HF), jnp.float32),
      ],
  )


def _segsum_kernel(h, sd):
  return _segsum_kernel_built()(h, sd)


def _proj_body(x_ref, w_ref, b_ref, o_ref):
  h = jnp.dot(x_ref[...], w_ref[...], preferred_element_type=jnp.float32)
  o_ref[...] = h + b_ref[...]


def _project(x, w, b):
  n, k = x.shape
  grid = n // ROWB
  return pl.pallas_call(
      _proj_body,
      grid=(grid,),
      in_specs=[
          pl.BlockSpec((ROWB, k), lambda i: (i, 0)),
          pl.BlockSpec((k, HF), lambda i: (0, 0)),
          pl.BlockSpec((1, HF), lambda i: (0, 0)),
      ],
      out_specs=pl.BlockSpec((ROWB, HF), lambda i: (i, 0)),
      out_shape=jax.ShapeDtypeStruct((n, HF), jnp.float32),
  )(x, w, b.reshape(1, HF))


def _layer_body(h_ref, a1_ref, a2_ref, c1_ref, c2_ref,
                root_ref, b1_ref, b2_ref, bias_ref, o_ref):
  acc = jnp.dot(h_ref[...], root_ref[...], preferred_element_type=jnp.float32)

  r1 = 1.0 / jnp.maximum(c1_ref[0, 0], 1.0)
  r2 = 1.0 / jnp.maximum(c2_ref[0, 0], 1.0)

  acc += jnp.dot(a1_ref[0] * r1[:, None], b1_ref[0],
                 preferred_element_type=jnp.float32)
  acc += jnp.dot(a2_ref[0] * r2[:, None], b2_ref[0],
                 preferred_element_type=jnp.float32)

  acc += bias_ref[...]
  o_ref[...] = jnp.maximum(acc, 0.0)


def _layer(h, a, cnt, root, b1, b2, bias):
  grid = N_ALL // ROWB
  cnt_r = cnt[:, :N_U].reshape(NREL * NBLK_U, 1, ROWB)

  def iu(i):
    return jnp.where(i < NBLK_U, i, i - NBLK_U)

  def rel1(i):
    return jnp.where(i < NBLK_U, 1, 0)

  def rel2(i):
    return jnp.where(i < NBLK_U, 2, 3)

  def dom(i):
    return jnp.where(i < NBLK_U, 0, 1)

  return pl.pallas_call(
      _layer_body,
      grid=(grid,),
      in_specs=[
          pl.BlockSpec((ROWB, HF), lambda i: (i, 0)),
          pl.BlockSpec((1, ROWB, HF), lambda i: (rel1(i), iu(i), 0)),
          pl.BlockSpec((1, ROWB, HF), lambda i: (rel2(i), iu(i), 0)),
          pl.BlockSpec((1, 1, ROWB), lambda i: (rel1(i) * NBLK_U + iu(i), 0, 0)),
          pl.BlockSpec((1, 1, ROWB), lambda i: (rel2(i) * NBLK_U + iu(i), 0, 0)),
          pl.BlockSpec((HF, HF), lambda i: (0, 0)),
          pl.BlockSpec((1, HF, HF), lambda i: (dom(i), 0, 0)),
          pl.BlockSpec((1, HF, HF), lambda i: (dom(i), 0, 0)),
          pl.BlockSpec((1, HF), lambda i: (0, 0)),
      ],
      out_specs=pl.BlockSpec((ROWB, HF), lambda i: (i, 0)),
      out_shape=jax.ShapeDtypeStruct((N_ALL, HF), jnp.float32),
  )(h, a, a, cnt_r, cnt_r, root, b1, b2, bias.reshape(1, HF))


def kernel(x_user, x_item, edge_index_rates, edge_index_rated_by,
           edge_index_follows, edge_index_similar, lin_user_W, lin_user_b,
           lin_item_W, lin_item_b, basis_0, comp_0, root_0, bias_0,
           basis_1, comp_1, root_1, bias_1):
  pad = E_PAD - NE
  pad_src = (jnp.arange(pad, dtype=jnp.int32) % 64)
  pad_dst = N_U + (jnp.arange(pad, dtype=jnp.int32) % (OUT_ROWS - N_U))

  def prep(ei, src_off):
    s = jnp.concatenate([ei[0] + src_off, pad_src])
    d = jnp.concatenate([ei[1], pad_dst])
    return s, d

  s0, d0 = prep(edge_index_rates, 0)
  s1, d1 = prep(edge_index_rated_by, N_U)
  s2, d2 = prep(edge_index_follows, 0)
  s3, d3 = prep(edge_index_similar, N_U)
  src_g = jnp.concatenate([s0, s1, s2, s3]).astype(jnp.int32)
  dst_g = jnp.concatenate([d0, d1, d2, d3]).astype(jnp.int32)
  sd = jnp.stack([src_g.reshape(NREL * NCHUNK, CHUNK),
                  dst_g.reshape(NREL * NCHUNK, CHUNK)], axis=1)

  hu = _project(x_user, lin_user_W, lin_user_b)
  hi = _project(x_item, lin_item_W, lin_item_b)
  h = jnp.concatenate([hu, hi], axis=0)

  cnt = _count_kernel(dst_g)

  def weights(basis, comp):
    b = basis[0]
    b1 = jnp.stack([comp[1, 0] * b, comp[0, 0] * b])
    b2 = jnp.stack([comp[2, 0] * b, comp[3, 0] * b])
    return b1, b2

  b1_0, b2_0 = weights(basis_0, comp_0)
  b1_1, b2_1 = weights(basis_1, comp_1)

  for root, b1, b2, bias in ((root_0, b1_0, b2_0, bias_0),
                             (root_1, b1_1, b2_1, bias_1)):
    a = _segsum_kernel(h, sd)
    h = _layer(h, a, cnt, root, b1, b2, bias)

  return h[:N_U], h[N_U:]


# speedup vs baseline: 6.5677x; 1.0002x over previous
import functools

import jax
import jax.numpy as jnp
from jax import lax
from jax.experimental import pallas as pl
from jax.experimental.pallas import tpu as pltpu
from jax.experimental.pallas import tpu_sc as plsc

N_U = 25000
N_ALL = 2 * N_U
HF = 128
NREL = 4
NE = 150000
CHUNK = 96
NCHUNK = 1568
E_PAD = NCHUNK * CHUNK
NTILE = 16
NCORE = 2
CPT = NCHUNK // NTILE
HALF = 12544
ACC_ROWS = 12800
OUT_ROWS = 2 * HALF
RPT = HALF // NTILE
WB = 56
ZCH = 16
CNT_ROWS = 25600
ROWB = 1000
NBLK_U = N_U // ROWB


@functools.cache
def _sc_mesh():
  return plsc.VectorSubcoreMesh(core_axis_name="c", subcore_axis_name="s",
                                num_cores=NCORE, num_subcores=NTILE)


def _zero_fill(ref, rows, cols):
  zv = jnp.zeros((16,), jnp.float32)
  per_row = cols // 16

  def body(k, _):
    i = k // per_row
    j = k % per_row
    ref[i, pl.ds(j * 16, 16)] = zv
    return 0

  lax.fori_loop(0, rows * per_row, body, 0)


def _zero_fill_1d(ref, n, value=0.0):
  v = jnp.full((16,), value, jnp.float32)

  def body(k, _):
    ref[pl.ds(k * 16, 16)] = v
    return 0

  lax.fori_loop(0, n // 16, body, 0)


def _count_body(dst_hbm, cnt_hbm, idx_v, ones_v, zb_v, acc):
  c = lax.axis_index("c")
  t = lax.axis_index("s")

  _zero_fill_1d(zb_v, 3200, 0.0)
  _zero_fill_1d(ones_v, CHUNK, 1.0)

  pltpu.sync_copy(zb_v, acc.at[pl.ds(pl.multiple_of(t * 3200, 8), 3200)])
  plsc.subcore_barrier()

  for r_local in range(2):
    rel_off = r_local * CNT_ROWS

    def chunk_body(j, _):
      tile_off = pl.multiple_of((t * CPT + j) * CHUNK, CHUNK)
      for r_glob in range(NREL):
        @pl.when(c * 2 + r_local == r_glob)
        def _():
          pltpu.sync_copy(
              dst_hbm.at[pl.ds(r_glob * E_PAD + tile_off, CHUNK)], idx_v)

      def shift(k, _):
        idx_v[pl.ds(k * 16, 16)] = idx_v[pl.ds(k * 16, 16)] + rel_off
        return 0

      lax.fori_loop(0, CHUNK // 16, shift, 0)
      pltpu.sync_copy(ones_v, acc.at[idx_v], add=True)
      return 0

    lax.fori_loop(0, CPT, chunk_body, 0)

  plsc.subcore_barrier()
  for r_local in range(2):
    toff = pl.multiple_of(t * 1600, 8)
    pltpu.sync_copy(
        acc.at[pl.ds(r_local * CNT_ROWS + toff, 1600)],
        zb_v.at[pl.ds(0, 1600)],
    )
    for r_glob in range(NREL):
      @pl.when(c * 2 + r_local == r_glob)
      def _():
        pltpu.sync_copy(
            zb_v.at[pl.ds(0, 1600)],
            cnt_hbm.at[pl.ds(r_glob * CNT_ROWS + toff, 1600)],
        )


@functools.cache
def _count_kernel_built():
  return pl.kernel(
      _count_body,
      out_type=jax.ShapeDtypeStruct((NREL * CNT_ROWS,), jnp.float32),
      mesh=_sc_mesh(),
      scratch_types=[
          pltpu.VMEM((CHUNK,), jnp.int32),
          pltpu.VMEM((CHUNK,), jnp.float32),
          pltpu.VMEM((3200,), jnp.float32),
          pltpu.VMEM_SHARED((2 * CNT_ROWS,), jnp.float32),
      ],
  )


def _count_kernel(dst_g):
  return _count_kernel_built()(dst_g).reshape(NREL, CNT_ROWS)


def _segsum_body(h_hbm, sd_hbm, a_hbm,
                 idx_v, rows_v, zb_v, si0, si1, si2, si3, sg0, sg1, acc):
  c = lax.axis_index("c")
  t = lax.axis_index("s")
  base = c * HALF
  isems = (si0, si1, si2, si3)
  gsems = (sg0, sg1)

  _zero_fill(zb_v, ZCH, HF)

  for r in range(NREL):
    for z in range(800 // ZCH):
      pltpu.sync_copy(
          zb_v, acc.at[pl.ds(pl.multiple_of(t * 800 + z * ZCH, 8), ZCH)])
    plsc.subcore_barrier()

    def idx_start(j):
      chunk_id = r * NCHUNK + t * CPT + j
      for sl in range(4):
        @pl.when((j & 3) == sl)
        def _(sl=sl):
          pltpu.async_copy(sd_hbm.at[chunk_id], idx_v.at[sl], isems[sl])

    def gather_start(j):
      for sl in range(4):
        @pl.when((j & 3) == sl)
        def _(sl=sl):
          chunk_id = r * NCHUNK + t * CPT + j
          pltpu.make_async_copy(sd_hbm.at[chunk_id], idx_v.at[sl],
                                isems[sl]).wait()

          def remap(k, _):
            d = idx_v[sl, 1, pl.ds(k * 16, 16)]
            local = d - base
            ok = (local >= 0) & (local < HALF)
            trash = HALF + (d & 255)
            idx_v[sl, 1, pl.ds(k * 16, 16)] = jnp.where(ok, local, trash)
            return 0

          lax.fori_loop(0, CHUNK // 16, remap, 0)
          for b in range(2):
            @pl.when((j & 1) == b)
            def _(sl=sl, b=b):
              pltpu.async_copy(h_hbm.at[idx_v.at[sl, 0]], rows_v.at[b],
                               gsems[b])

    def finish(j):
      for sl in range(4):
        for b in range(2):
          @pl.when(((j & 3) == sl) & ((j & 1) == b))
          def _(sl=sl, b=b):
            pltpu.make_async_copy(h_hbm.at[idx_v.at[sl, 0]], rows_v.at[b],
                                  gsems[b]).wait()
            pltpu.sync_copy(rows_v.at[b], acc.at[idx_v.at[sl, 1]], add=True)

    idx_start(0)
    idx_start(1)
    idx_start(2)

    def chunk_body(j, _):
      gather_start(j)

      @pl.when(j > 0)
      def _():
        finish(j - 1)

      @pl.when(j + 3 < CPT)
      def _():
        idx_start(j + 3)

      return 0

    lax.fori_loop(0, CPT, chunk_body, 0)
    finish(CPT - 1)
    plsc.subcore_barrier()

    for w in range(RPT // WB):
      row0 = pl.multiple_of(t * RPT + w * WB, 8)
      pltpu.sync_copy(acc.at[pl.ds(row0, WB)], rows_v.at[0, pl.ds(0, WB)])
      pltpu.sync_copy(rows_v.at[0, pl.ds(0, WB)],
                      a_hbm.at[r, pl.ds(pl.multiple_of(base + row0, 8), WB)])

    plsc.subcore_barrier()


@functools.cache
def _segsum_kernel_built():
  return pl.kernel(
      _segsum_body,
      out_type=jax.ShapeDtypeStruct((NREL, OUT_ROWS, HF), jnp.float32),
      mesh=_sc_mesh(),
      scratch_types=[
          pltpu.VMEM((4, 2, CHUNK), jnp.int32),
          pltpu.VMEM((2, CHUNK, HF), jnp.float32),
          pltpu.VMEM((ZCH, HF), jnp.float32),
          pltpu.SemaphoreType.DMA,
          pltpu.SemaphoreType.DMA,
          pltpu.SemaphoreType.DMA,
          pltpu.SemaphoreType.DMA,
          pltpu.SemaphoreType.DMA,
          pltpu.SemaphoreType.DMA,
          pltpu.VMEM_SHARED((ACC_ROWS, HF), jnp.float32),
      ],
  )


def _segsum_kernel(h, sd):
  return _segsum_kernel_built()(h, sd)


def _proj_body(x_ref, w_ref, b_ref, o_ref):
  h = jnp.dot(x_ref[...], w_ref[...], preferred_element_type=jnp.float32)
  o_ref[...] = h + b_ref[...]


def _project(x, w, b):
  n, k = x.shape
  grid = n // ROWB
  return pl.pallas_call(
      _proj_body,
      grid=(grid,),
      in_specs=[
          pl.BlockSpec((ROWB, k), lambda i: (i, 0)),
          pl.BlockSpec((k, HF), lambda i: (0, 0)),
          pl.BlockSpec((1, HF), lambda i: (0, 0)),
      ],
      out_specs=pl.BlockSpec((ROWB, HF), lambda i: (i, 0)),
      out_shape=jax.ShapeDtypeStruct((n, HF), jnp.float32),
  )(x, w, b.reshape(1, HF))


def _layer_body(h_ref, a1_ref, a2_ref, c1_ref, c2_ref,
                root_ref, b1_ref, b2_ref, bias_ref, o_ref):
  acc = jnp.dot(h_ref[...], root_ref[...], preferred_element_type=jnp.float32)

  r1 = 1.0 / jnp.maximum(c1_ref[0, 0], 1.0)
  r2 = 1.0 / jnp.maximum(c2_ref[0, 0], 1.0)

  acc += jnp.dot(a1_ref[0] * r1[:, None], b1_ref[0],
                 preferred_element_type=jnp.float32)
  acc += jnp.dot(a2_ref[0] * r2[:, None], b2_ref[0],
                 preferred_element_type=jnp.float32)

  acc += bias_ref[...]
  o_ref[...] = jnp.maximum(acc, 0.0)


def _layer(h, a, cnt, root, b1, b2, bias):
  grid = N_ALL // ROWB
  cnt_r = cnt[:, :N_U].reshape(NREL * NBLK_U, 1, ROWB)

  def iu(i):
    return jnp.where(i < NBLK_U, i, i - NBLK_U)

  def rel1(i):
    return jnp.where(i < NBLK_U, 1, 0)

  def rel2(i):
    return jnp.where(i < NBLK_U, 2, 3)

  def dom(i):
    return jnp.where(i < NBLK_U, 0, 1)

  return pl.pallas_call(
      _layer_body,
      grid=(grid,),
      in_specs=[
          pl.BlockSpec((ROWB, HF), lambda i: (i, 0)),
          pl.BlockSpec((1, ROWB, HF), lambda i: (rel1(i), iu(i), 0)),
          pl.BlockSpec((1, ROWB, HF), lambda i: (rel2(i), iu(i), 0)),
          pl.BlockSpec((1, 1, ROWB), lambda i: (rel1(i) * NBLK_U + iu(i), 0, 0)),
          pl.BlockSpec((1, 1, ROWB), lambda i: (rel2(i) * NBLK_U + iu(i), 0, 0)),
          pl.BlockSpec((HF, HF), lambda i: (0, 0)),
          pl.BlockSpec((1, HF, HF), lambda i: (dom(i), 0, 0)),
          pl.BlockSpec((1, HF, HF), lambda i: (dom(i), 0, 0)),
          pl.BlockSpec((1, HF), lambda i: (0, 0)),
      ],
      out_specs=pl.BlockSpec((ROWB, HF), lambda i: (i, 0)),
      out_shape=jax.ShapeDtypeStruct((N_ALL, HF), jnp.float32),
  )(h, a, a, cnt_r, cnt_r, root, b1, b2, bias.reshape(1, HF))


def kernel(x_user, x_item, edge_index_rates, edge_index_rated_by,
           edge_index_follows, edge_index_similar, lin_user_W, lin_user_b,
           lin_item_W, lin_item_b, basis_0, comp_0, root_0, bias_0,
           basis_1, comp_1, root_1, bias_1):
  pad = E_PAD - NE
  pad_src = (jnp.arange(pad, dtype=jnp.int32) % 64)
  pad_dst = N_U + (jnp.arange(pad, dtype=jnp.int32) % (OUT_ROWS - N_U))

  def prep(ei, src_off):
    s = jnp.concatenate([ei[0] + src_off, pad_src])
    d = jnp.concatenate([ei[1], pad_dst])
    return s, d

  s0, d0 = prep(edge_index_rates, 0)
  s1, d1 = prep(edge_index_rated_by, N_U)
  s2, d2 = prep(edge_index_follows, 0)
  s3, d3 = prep(edge_index_similar, N_U)
  src_g = jnp.concatenate([s0, s1, s2, s3]).astype(jnp.int32)
  dst_g = jnp.concatenate([d0, d1, d2, d3]).astype(jnp.int32)
  sd = jnp.stack([src_g.reshape(NREL * NCHUNK, CHUNK),
                  dst_g.reshape(NREL * NCHUNK, CHUNK)], axis=1)

  hu = _project(x_user, lin_user_W, lin_user_b)
  hi = _project(x_item, lin_item_W, lin_item_b)
  h = jnp.concatenate([hu, hi], axis=0)

  cnt = _count_kernel(dst_g)

  def weights(basis, comp):
    b = basis[0]
    b1 = jnp.stack([comp[1, 0] * b, comp[0, 0] * b])
    b2 = jnp.stack([comp[2, 0] * b, comp[3, 0] * b])
    return b1, b2

  b1_0, b2_0 = weights(basis_0, comp_0)
  b1_1, b2_1 = weights(basis_1, comp_1)

  for root, b1, b2, bias in ((root_0, b1_0, b2_0, bias_0),
                             (root_1, b1_1, b2_1, bias_1)):
    a = _segsum_kernel(h, sd)
    h = _layer(h, a, cnt, root, b1, b2, bias)

  return h[:N_U], h[N_U:]


# speedup vs baseline: 6.6925x; 1.0190x over previous
import functools

import jax
import jax.numpy as jnp
from jax import lax
from jax.experimental import pallas as pl
from jax.experimental.pallas import tpu as pltpu
from jax.experimental.pallas import tpu_sc as plsc

N_U = 25000
N_ALL = 2 * N_U
HF = 128
NREL = 4
NE = 150000
CHUNK = 96
NCHUNK = 1568
E_PAD = NCHUNK * CHUNK
NTILE = 16
NCORE = 2
CPT = NCHUNK // NTILE
HALF = 12544
ACC_ROWS = 12800
OUT_ROWS = 2 * HALF
RPT = HALF // NTILE
WB = 56
ZCH = 16
CNT_ROWS = 25600
ROWB = 1000
NBLK_U = N_U // ROWB


@functools.cache
def _sc_mesh():
  return plsc.VectorSubcoreMesh(core_axis_name="c", subcore_axis_name="s",
                                num_cores=NCORE, num_subcores=NTILE)


def _zero_fill(ref, rows, cols):
  zv = jnp.zeros((16,), jnp.float32)
  per_row = cols // 16

  def body(k, _):
    i = k // per_row
    j = k % per_row
    ref[i, pl.ds(j * 16, 16)] = zv
    return 0

  lax.fori_loop(0, rows * per_row, body, 0)


def _zero_fill_1d(ref, n, value=0.0):
  v = jnp.full((16,), value, jnp.float32)

  def body(k, _):
    ref[pl.ds(k * 16, 16)] = v
    return 0

  lax.fori_loop(0, n // 16, body, 0)


def _count_body(dst_hbm, cnt_hbm, idx_v, ones_v, zb_v,
                si0, si1, si2, si3, acc):
  c = lax.axis_index("c")
  t = lax.axis_index("s")
  isems = (si0, si1, si2, si3)

  _zero_fill_1d(zb_v, 3200, 0.0)
  _zero_fill_1d(ones_v, CHUNK, 1.0)

  pltpu.sync_copy(zb_v, acc.at[pl.ds(pl.multiple_of(t * 3200, 8), 3200)])
  plsc.subcore_barrier()

  tot = 2 * CPT

  def chunk_off(j):
    sel = (j >= CPT).astype(jnp.int32)
    jj = j - sel * CPT
    off = (c * 2 + sel) * E_PAD + (t * CPT + jj) * CHUNK
    return sel, pl.multiple_of(off, 8)

  def idx_start(j):
    _, off = chunk_off(j)
    for sl in range(4):
      @pl.when((j & 3) == sl)
      def _(sl=sl):
        pltpu.async_copy(dst_hbm.at[pl.ds(off, CHUNK)], idx_v.at[sl],
                         isems[sl])

  def consume(j):
    sel, _ = chunk_off(j)
    rel_off = sel * CNT_ROWS
    for sl in range(4):
      @pl.when((j & 3) == sl)
      def _(sl=sl):
        pltpu.make_async_copy(dst_hbm.at[pl.ds(0, CHUNK)], idx_v.at[sl],
                              isems[sl]).wait()

        def shift(k, _):
          idx_v[sl, pl.ds(k * 16, 16)] = (
              idx_v[sl, pl.ds(k * 16, 16)] + rel_off)
          return 0

        lax.fori_loop(0, CHUNK // 16, shift, 0)
        pltpu.sync_copy(ones_v, acc.at[idx_v.at[sl]], add=True)

  idx_start(jnp.int32(0))
  idx_start(jnp.int32(1))
  idx_start(jnp.int32(2))

  def chunk_body(j, _):
    consume(j)

    @pl.when(j + 3 < tot)
    def _():
      idx_start(j + 3)

    return 0

  lax.fori_loop(0, tot, chunk_body, 0)
  plsc.subcore_barrier()
  for r_local in range(2):
    toff = pl.multiple_of(t * 1600, 8)
    pltpu.sync_copy(
        acc.at[pl.ds(r_local * CNT_ROWS + toff, 1600)],
        zb_v.at[pl.ds(0, 1600)],
    )
    for r_glob in range(NREL):
      @pl.when(c * 2 + r_local == r_glob)
      def _():
        pltpu.sync_copy(
            zb_v.at[pl.ds(0, 1600)],
            cnt_hbm.at[pl.ds(r_glob * CNT_ROWS + toff, 1600)],
        )


@functools.cache
def _count_kernel_built():
  return pl.kernel(
      _count_body,
      out_type=jax.ShapeDtypeStruct((NREL * CNT_ROWS,), jnp.float32),
      mesh=_sc_mesh(),
      scratch_types=[
          pltpu.VMEM((4, CHUNK), jnp.int32),
          pltpu.VMEM((CHUNK,), jnp.float32),
          pltpu.VMEM((3200,), jnp.float32),
          pltpu.SemaphoreType.DMA,
          pltpu.SemaphoreType.DMA,
          pltpu.SemaphoreType.DMA,
          pltpu.SemaphoreType.DMA,
          pltpu.VMEM_SHARED((2 * CNT_ROWS,), jnp.float32),
      ],
  )


def _count_kernel(dst_g):
  return _count_kernel_built()(dst_g).reshape(NREL, CNT_ROWS)


def _segsum_body(h_hbm, sd_hbm, a_hbm,
                 idx_v, rows_v, zb_v, si0, si1, si2, si3, sg0, sg1, acc):
  c = lax.axis_index("c")
  t = lax.axis_index("s")
  base = c * HALF
  isems = (si0, si1, si2, si3)
  gsems = (sg0, sg1)

  _zero_fill(zb_v, ZCH, HF)

  for r in range(NREL):
    for z in range(800 // ZCH):
      pltpu.sync_copy(
          zb_v, acc.at[pl.ds(pl.multiple_of(t * 800 + z * ZCH, 8), ZCH)])
    plsc.subcore_barrier()

    def idx_start(j):
      chunk_id = r * NCHUNK + t * CPT + j
      for sl in range(4):
        @pl.when((j & 3) == sl)
        def _(sl=sl):
          pltpu.async_copy(sd_hbm.at[chunk_id], idx_v.at[sl], isems[sl])

    def gather_start(j):
      for sl in range(4):
        @pl.when((j & 3) == sl)
        def _(sl=sl):
          chunk_id = r * NCHUNK + t * CPT + j
          pltpu.make_async_copy(sd_hbm.at[chunk_id], idx_v.at[sl],
                                isems[sl]).wait()

          def remap(k, _):
            d = idx_v[sl, 1, pl.ds(k * 16, 16)]
            local = d - base
            ok = (local >= 0) & (local < HALF)
            trash = HALF + (d & 255)
            idx_v[sl, 1, pl.ds(k * 16, 16)] = jnp.where(ok, local, trash)
            return 0

          lax.fori_loop(0, CHUNK // 16, remap, 0)
          for b in range(2):
            @pl.when((j & 1) == b)
            def _(sl=sl, b=b):
              pltpu.async_copy(h_hbm.at[idx_v.at[sl, 0]], rows_v.at[b],
                               gsems[b])

    def finish(j):
      for sl in range(4):
        for b in range(2):
          @pl.when(((j & 3) == sl) & ((j & 1) == b))
          def _(sl=sl, b=b):
            pltpu.make_async_copy(h_hbm.at[idx_v.at[sl, 0]], rows_v.at[b],
                                  gsems[b]).wait()
            pltpu.sync_copy(rows_v.at[b], acc.at[idx_v.at[sl, 1]], add=True)

    idx_start(0)
    idx_start(1)
    idx_start(2)

    def chunk_body(j, _):
      gather_start(j)

      @pl.when(j > 0)
      def _():
        finish(j - 1)

      @pl.when(j + 3 < CPT)
      def _():
        idx_start(j + 3)

      return 0

    lax.fori_loop(0, CPT, chunk_body, 0)
    finish(CPT - 1)
    plsc.subcore_barrier()

    for w in range(RPT // WB):
      row0 = pl.multiple_of(t * RPT + w * WB, 8)
      pltpu.sync_copy(acc.at[pl.ds(row0, WB)], rows_v.at[0, pl.ds(0, WB)])
      pltpu.sync_copy(rows_v.at[0, pl.ds(0, WB)],
                      a_hbm.at[r, pl.ds(pl.multiple_of(base + row0, 8), WB)])

    plsc.subcore_barrier()


@functools.cache
def _segsum_kernel_built():
  return pl.kernel(
      _segsum_body,
      out_type=jax.ShapeDtypeStruct((NREL, OUT_ROWS, HF), jnp.float32),
      mesh=_sc_mesh(),
      scratch_types=[
          pltpu.VMEM((4, 2, CHUNK), jnp.int32),
          pltpu.VMEM((2, CHUNK, HF), jnp.float32),
          pltpu.VMEM((ZCH, HF), jnp.float32),
          pltpu.SemaphoreType.DMA,
          pltpu.SemaphoreType.DMA,
          pltpu.SemaphoreType.DMA,
          pltpu.SemaphoreType.DMA,
          pltpu.SemaphoreType.DMA,
          pltpu.SemaphoreType.DMA,
          pltpu.VMEM_SHARED((ACC_ROWS, HF), jnp.float32),
      ],
  )


def _segsum_kernel(h, sd):
  return _segsum_kernel_built()(h, sd)


def _proj_body(x_ref, w_ref, b_ref, o_ref):
  h = jnp.dot(x_ref[...], w_ref[...], preferred_element_type=jnp.float32)
  o_ref[...] = h + b_ref[...]


def _project(x, w, b):
  n, k = x.shape
  grid = n // ROWB
  return pl.pallas_call(
      _proj_body,
      grid=(grid,),
      in_specs=[
          pl.BlockSpec((ROWB, k), lambda i: (i, 0)),
          pl.BlockSpec((k, HF), lambda i: (0, 0)),
          pl.BlockSpec((1, HF), lambda i: (0, 0)),
      ],
      out_specs=pl.BlockSpec((ROWB, HF), lambda i: (i, 0)),
      out_shape=jax.ShapeDtypeStruct((n, HF), jnp.float32),
  )(x, w, b.reshape(1, HF))


def _layer_body(h_ref, a1_ref, a2_ref, c1_ref, c2_ref,
                root_ref, b1_ref, b2_ref, bias_ref, o_ref):
  acc = jnp.dot(h_ref[...], root_ref[...], preferred_element_type=jnp.float32)

  r1 = 1.0 / jnp.maximum(c1_ref[0, 0], 1.0)
  r2 = 1.0 / jnp.maximum(c2_ref[0, 0], 1.0)

  acc += jnp.dot(a1_ref[0] * r1[:, None], b1_ref[0],
                 preferred_element_type=jnp.float32)
  acc += jnp.dot(a2_ref[0] * r2[:, None], b2_ref[0],
                 preferred_element_type=jnp.float32)

  acc += bias_ref[...]
  o_ref[...] = jnp.maximum(acc, 0.0)


def _layer(h, a, cnt, root, b1, b2, bias):
  grid = N_ALL // ROWB
  cnt_r = cnt[:, :N_U].reshape(NREL * NBLK_U, 1, ROWB)

  def iu(i):
    return jnp.where(i < NBLK_U, i, i - NBLK_U)

  def rel1(i):
    return jnp.where(i < NBLK_U, 1, 0)

  def rel2(i):
    return jnp.where(i < NBLK_U, 2, 3)

  def dom(i):
    return jnp.where(i < NBLK_U, 0, 1)

  return pl.pallas_call(
      _layer_body,
      grid=(grid,),
      in_specs=[
          pl.BlockSpec((ROWB, HF), lambda i: (i, 0)),
          pl.BlockSpec((1, ROWB, HF), lambda i: (rel1(i), iu(i), 0)),
          pl.BlockSpec((1, ROWB, HF), lambda i: (rel2(i), iu(i), 0)),
          pl.BlockSpec((1, 1, ROWB), lambda i: (rel1(i) * NBLK_U + iu(i), 0, 0)),
          pl.BlockSpec((1, 1, ROWB), lambda i: (rel2(i) * NBLK_U + iu(i), 0, 0)),
          pl.BlockSpec((HF, HF), lambda i: (0, 0)),
          pl.BlockSpec((1, HF, HF), lambda i: (dom(i), 0, 0)),
          pl.BlockSpec((1, HF, HF), lambda i: (dom(i), 0, 0)),
          pl.BlockSpec((1, HF), lambda i: (0, 0)),
      ],
      out_specs=pl.BlockSpec((ROWB, HF), lambda i: (i, 0)),
      out_shape=jax.ShapeDtypeStruct((N_ALL, HF), jnp.float32),
  )(h, a, a, cnt_r, cnt_r, root, b1, b2, bias.reshape(1, HF))


def kernel(x_user, x_item, edge_index_rates, edge_index_rated_by,
           edge_index_follows, edge_index_similar, lin_user_W, lin_user_b,
           lin_item_W, lin_item_b, basis_0, comp_0, root_0, bias_0,
           basis_1, comp_1, root_1, bias_1):
  pad = E_PAD - NE
  pad_src = (jnp.arange(pad, dtype=jnp.int32) % 64)
  pad_dst = N_U + (jnp.arange(pad, dtype=jnp.int32) % (OUT_ROWS - N_U))

  def prep(ei, src_off):
    s = jnp.concatenate([ei[0] + src_off, pad_src])
    d = jnp.concatenate([ei[1], pad_dst])
    return s, d

  s0, d0 = prep(edge_index_rates, 0)
  s1, d1 = prep(edge_index_rated_by, N_U)
  s2, d2 = prep(edge_index_follows, 0)
  s3, d3 = prep(edge_index_similar, N_U)
  src_g = jnp.concatenate([s0, s1, s2, s3]).astype(jnp.int32)
  dst_g = jnp.concatenate([d0, d1, d2, d3]).astype(jnp.int32)
  sd = jnp.stack([src_g.reshape(NREL * NCHUNK, CHUNK),
                  dst_g.reshape(NREL * NCHUNK, CHUNK)], axis=1)

  hu = _project(x_user, lin_user_W, lin_user_b)
  hi = _project(x_item, lin_item_W, lin_item_b)
  h = jnp.concatenate([hu, hi], axis=0)

  cnt = _count_kernel(dst_g)

  def weights(basis, comp):
    b = basis[0]
    b1 = jnp.stack([comp[1, 0] * b, comp[0, 0] * b])
    b2 = jnp.stack([comp[2, 0] * b, comp[3, 0] * b])
    return b1, b2

  b1_0, b2_0 = weights(basis_0, comp_0)
  b1_1, b2_1 = weights(basis_1, comp_1)

  for root, b1, b2, bias in ((root_0, b1_0, b2_0, bias_0),
                             (root_1, b1_1, b2_1, bias_1)):
    a = _segsum_kernel(h, sd)
    h = _layer(h, a, cnt, root, b1, b2, bias)

  return h[:N_U], h[N_U:]
